# baseline (speedup 1.0000x reference)
import jax
import jax.numpy as jnp
from jax.experimental import pallas as pl

_NUM_AGENTS = 4096
_FEAT = 3
_LANES = 128
_ROWS_IN = 8192 * _FEAT // _LANES
_ROWS_OUT = _NUM_AGENTS * _FEAT // _LANES


def _slice_body(in_ref, out_ref):
    out_ref[...] = in_ref[...]


def kernel(pos_phi, num_agents):
    flat = jnp.reshape(pos_phi, (_ROWS_IN, _LANES))
    out = pl.pallas_call(
        _slice_body,
        out_shape=jax.ShapeDtypeStruct((_ROWS_OUT, _LANES), jnp.float32),
        grid=(1,),
        in_specs=[pl.BlockSpec((_ROWS_OUT, _LANES), lambda i: (0, 0))],
        out_specs=pl.BlockSpec((_ROWS_OUT, _LANES), lambda i: (0, 0)),
    )(flat)
    return jnp.reshape(out, (_NUM_AGENTS, _FEAT))
